# baseline (speedup 1.0000x reference)
import functools

import jax
import jax.numpy as jnp
from jax import lax
from jax.experimental import pallas as pl
from jax.experimental.pallas import tpu as pltpu
from jax.experimental.pallas import tpu_sc as plsc

NC = 2
NS = 16
NW = NC * NS

CH = 128
GPB = 4


def _emb_lookup(table, idx3, *, n_per_w, d):
    K = n_per_w // CH
    T = K // GPB
    rows_chunk = GPB * CH

    mesh = plsc.VectorSubcoreMesh(core_axis_name="c", subcore_axis_name="s")

    @functools.partial(
        pl.kernel,
        out_type=jax.ShapeDtypeStruct((NW * n_per_w, d), jnp.float32),
        mesh=mesh,
        scratch_types=[
            pltpu.VMEM((K, CH), jnp.int32),
            pltpu.VMEM((2, rows_chunk, d), jnp.float32),
            pltpu.SemaphoreType.DMA,
        ],
        compiler_params=pltpu.CompilerParams(use_tc_tiling_on_sc=False),
    )
    def body(table_hbm, idx_hbm, out_hbm, idx_v, rows_v, gsem):
        wid = lax.axis_index("s") * NC + lax.axis_index("c")
        base = wid * n_per_w
        pltpu.sync_copy(idx_hbm.at[wid], idx_v)

        def fire(t, buf):
            for g in range(GPB):
                pltpu.async_copy(
                    table_hbm.at[idx_v.at[t * GPB + g]],
                    rows_v.at[buf, pl.ds(g * CH, CH)],
                    gsem,
                )

        def drain(buf):
            for g in range(GPB):
                pltpu.make_async_copy(
                    table_hbm.at[idx_v.at[g]],
                    rows_v.at[buf, pl.ds(g * CH, CH)],
                    gsem,
                ).wait()

        fire(0, 0)

        @pl.loop(0, T - 1)
        def _(t):
            buf = lax.rem(t, 2)
            nxt = lax.rem(t + 1, 2)
            fire(t + 1, nxt)
            drain(buf)
            pltpu.sync_copy(
                rows_v.at[buf],
                out_hbm.at[pl.ds(base + t * rows_chunk, rows_chunk)],
            )

        last = (T - 1) % 2
        drain(last)
        pltpu.sync_copy(
            rows_v.at[last],
            out_hbm.at[pl.ds(base + (T - 1) * rows_chunk, rows_chunk)],
        )

    return body(table, idx3)


def kernel(emb_mat, input):
    v, d = emb_mat.shape
    b, h = input.shape
    n = b * h
    n_per_w = n // NW
    idx3 = input.reshape(NW, n_per_w // CH, CH).astype(jnp.int32)
    out = _emb_lookup(emb_mat, idx3, n_per_w=n_per_w, d=d)
    return out.reshape(b, h, d)
